# baseline (speedup 1.0000x reference)
import numpy as np
import jax
import jax.numpy as jnp
from jax.experimental import pallas as pl
from jax.experimental.pallas import tpu as pltpu

_B, _T, _P, _D = 8, 16, 576, 768
_N = _B * _T
_SPLIT = 2
_H = _P // _SPLIT
_NH = _N * _SPLIT

_WORDS = [0x8D744451, 0xB39A25C9, 0x587166EB, 0x27893CC9]
_FLAT = np.array([(w >> b) & 1 for w in _WORDS for b in range(32)], dtype=bool)
_HFLAT = np.repeat(_FLAT, _SPLIT)
_MASKED = np.nonzero(_HFLAT)[0]
_UNMASKED = np.nonzero(~_HFLAT)[0]
_NCP = len(_UNMASKED)
_NTOK = len(_MASKED)

_K = 16
_A = 8


def _body(x_ref, tok_ref, out_ref, tokf, ring, sem_g, sem_s, sem_t):
    tokf[...] = jnp.broadcast_to(tok_ref[0, :], (_H, _D))

    g = [
        pltpu.make_async_copy(x_ref.at[int(f)], ring.at[c % _K], sem_g)
        for c, f in enumerate(_UNMASKED)
    ]
    s = [
        pltpu.make_async_copy(ring.at[c % _K], out_ref.at[int(f)], sem_s)
        for c, f in enumerate(_UNMASKED)
    ]
    t = [pltpu.make_async_copy(tokf, out_ref.at[int(f)], sem_t) for f in _MASKED]

    for c in range(_A):
        g[c].start()
    waited_s = -1
    for c in range(_NCP):
        if c < _NTOK:
            t[c].start()
        g[c].wait()
        s[c].start()
        if c + _A < _NCP:
            if c + _A - _K >= 0:
                s[c + _A - _K].wait()
                waited_s = c + _A - _K
            g[c + _A].start()
    for c in range(waited_s + 1, _NCP):
        s[c].wait()
    for c in range(_NCP, _NTOK):
        t[c].start()
    for c in range(_NTOK):
        t[c].wait()


def kernel(x, mask_token):
    xh = x.reshape(_NH, _H, _D)
    tok = mask_token.reshape(1, _D)
    outh = pl.pallas_call(
        _body,
        in_specs=[
            pl.BlockSpec(memory_space=pl.ANY),
            pl.BlockSpec(memory_space=pltpu.VMEM),
        ],
        out_specs=pl.BlockSpec(memory_space=pl.ANY),
        out_shape=jax.ShapeDtypeStruct((_NH, _H, _D), x.dtype),
        scratch_shapes=[
            pltpu.VMEM((_H, _D), jnp.float32),
            pltpu.VMEM((_K, _H, _D), jnp.float32),
            pltpu.SemaphoreType.DMA,
            pltpu.SemaphoreType.DMA,
            pltpu.SemaphoreType.DMA,
        ],
    )(xh, tok)
    return outh.reshape(_B, _T, _P, _D)


# speedup vs baseline: 1.0156x; 1.0156x over previous
import numpy as np
import jax
import jax.numpy as jnp
from jax.experimental import pallas as pl
from jax.experimental.pallas import tpu as pltpu

_B, _T, _P, _D = 8, 16, 576, 768
_N = _B * _T

_WORDS = [0x8D744451, 0xB39A25C9, 0x587166EB, 0x27893CC9]
_FLAT = np.array([(w >> b) & 1 for w in _WORDS for b in range(32)], dtype=bool)
_MASKED = np.nonzero(_FLAT)[0]
_UNMASKED = np.nonzero(~_FLAT)[0]
_NCP = len(_UNMASKED)
_NTOK = len(_MASKED)

_K = 10
_A = 6


def _body(x_ref, tok_ref, out_ref, tokf, ring, sem_g, sem_s, sem_t):
    tokf[...] = jnp.broadcast_to(tok_ref[0, :], (_P, _D))

    g = [
        pltpu.make_async_copy(x_ref.at[int(f)], ring.at[c % _K], sem_g)
        for c, f in enumerate(_UNMASKED)
    ]
    s = [
        pltpu.make_async_copy(ring.at[c % _K], out_ref.at[int(f)], sem_s)
        for c, f in enumerate(_UNMASKED)
    ]
    t = [pltpu.make_async_copy(tokf, out_ref.at[int(f)], sem_t) for f in _MASKED]

    for c in range(_A):
        g[c].start()
    waited_s = -1
    for c in range(_NCP):
        if c < _NTOK:
            t[c].start()
        g[c].wait()
        s[c].start()
        if c + _A < _NCP:
            if c + _A - _K >= 0:
                s[c + _A - _K].wait()
                waited_s = c + _A - _K
            g[c + _A].start()
    for c in range(waited_s + 1, _NCP):
        s[c].wait()
    for c in range(_NCP, _NTOK):
        t[c].start()
    for c in range(_NTOK):
        t[c].wait()


def kernel(x, mask_token):
    x3 = x.reshape(_N, _P, _D)
    tok = mask_token.reshape(1, _D)
    out3 = pl.pallas_call(
        _body,
        in_specs=[
            pl.BlockSpec(memory_space=pl.ANY),
            pl.BlockSpec(memory_space=pltpu.VMEM),
        ],
        out_specs=pl.BlockSpec(memory_space=pl.ANY),
        out_shape=jax.ShapeDtypeStruct((_N, _P, _D), x.dtype),
        scratch_shapes=[
            pltpu.VMEM((_P, _D), jnp.float32),
            pltpu.VMEM((_K, _P, _D), jnp.float32),
            pltpu.SemaphoreType.DMA,
            pltpu.SemaphoreType.DMA,
            pltpu.SemaphoreType.DMA,
        ],
    )(x3, tok)
    return out3.reshape(_B, _T, _P, _D)


# speedup vs baseline: 1.0183x; 1.0026x over previous
import numpy as np
import jax
import jax.numpy as jnp
from jax.experimental import pallas as pl
from jax.experimental.pallas import tpu as pltpu

_B, _T, _P, _D = 8, 16, 576, 768
_N = _B * _T

_WORDS = [0x8D744451, 0xB39A25C9, 0x587166EB, 0x27893CC9]
_FLAT = np.array([(w >> b) & 1 for w in _WORDS for b in range(32)], dtype=bool)
_MASKED = np.nonzero(_FLAT)[0]
_UNMASKED = np.nonzero(~_FLAT)[0]
_NCP = len(_UNMASKED)
_NTOK = len(_MASKED)

_K = 12
_A = 8


def _body(x_ref, tok_ref, out_ref, tokf, ring, sem_g, sem_s, sem_t):
    tokf[...] = jnp.broadcast_to(tok_ref[0, :], (_P, _D))

    g = [
        pltpu.make_async_copy(x_ref.at[int(f)], ring.at[c % _K], sem_g)
        for c, f in enumerate(_UNMASKED)
    ]
    s = [
        pltpu.make_async_copy(ring.at[c % _K], out_ref.at[int(f)], sem_s)
        for c, f in enumerate(_UNMASKED)
    ]
    t = [pltpu.make_async_copy(tokf, out_ref.at[int(f)], sem_t) for f in _MASKED]

    for c in range(_A):
        g[c].start()
    waited_s = -1
    for c in range(_NCP):
        if c < _NTOK:
            t[c].start()
        g[c].wait()
        s[c].start()
        if c + _A < _NCP:
            if c + _A - _K >= 0:
                s[c + _A - _K].wait()
                waited_s = c + _A - _K
            g[c + _A].start()
    for c in range(waited_s + 1, _NCP):
        s[c].wait()
    for c in range(_NCP, _NTOK):
        t[c].start()
    for c in range(_NTOK):
        t[c].wait()


def kernel(x, mask_token):
    x3 = x.reshape(_N, _P, _D)
    tok = mask_token.reshape(1, _D)
    out3 = pl.pallas_call(
        _body,
        in_specs=[
            pl.BlockSpec(memory_space=pl.ANY),
            pl.BlockSpec(memory_space=pltpu.VMEM),
        ],
        out_specs=pl.BlockSpec(memory_space=pl.ANY),
        out_shape=jax.ShapeDtypeStruct((_N, _P, _D), x.dtype),
        scratch_shapes=[
            pltpu.VMEM((_P, _D), jnp.float32),
            pltpu.VMEM((_K, _P, _D), jnp.float32),
            pltpu.SemaphoreType.DMA,
            pltpu.SemaphoreType.DMA,
            pltpu.SemaphoreType.DMA,
        ],
    )(x3, tok)
    return out3.reshape(_B, _T, _P, _D)


# speedup vs baseline: 1.0313x; 1.0128x over previous
import numpy as np
import jax
import jax.numpy as jnp
from jax.experimental import pallas as pl
from jax.experimental.pallas import tpu as pltpu

_B, _T, _P, _D = 8, 16, 576, 768
_N = _B * _T

_WORDS = [0x8D744451, 0xB39A25C9, 0x587166EB, 0x27893CC9]
_FLAT = np.array([(w >> b) & 1 for w in _WORDS for b in range(32)], dtype=bool)
_MASKED = np.nonzero(_FLAT)[0]
_UNMASKED = np.nonzero(~_FLAT)[0]
_NCP = len(_UNMASKED)
_NTOK = len(_MASKED)

_K = 16
_A = 12


def _body(x_ref, tok_ref, out_ref, tokf, ring, sem_g, sem_s, sem_t):
    tokf[...] = jnp.broadcast_to(tok_ref[0, :], (_P, _D))

    g = [
        pltpu.make_async_copy(x_ref.at[int(f)], ring.at[c % _K], sem_g)
        for c, f in enumerate(_UNMASKED)
    ]
    s = [
        pltpu.make_async_copy(ring.at[c % _K], out_ref.at[int(f)], sem_s)
        for c, f in enumerate(_UNMASKED)
    ]
    t = [pltpu.make_async_copy(tokf, out_ref.at[int(f)], sem_t) for f in _MASKED]

    for c in range(_A):
        g[c].start()
    waited_s = -1
    for c in range(_NCP):
        if c < _NTOK:
            t[c].start()
        g[c].wait()
        s[c].start()
        if c + _A < _NCP:
            if c + _A - _K >= 0:
                s[c + _A - _K].wait()
                waited_s = c + _A - _K
            g[c + _A].start()
    for c in range(waited_s + 1, _NCP):
        s[c].wait()
    for c in range(_NCP, _NTOK):
        t[c].start()
    for c in range(_NTOK):
        t[c].wait()


def kernel(x, mask_token):
    x3 = x.reshape(_N, _P, _D)
    tok = mask_token.reshape(1, _D)
    out3 = pl.pallas_call(
        _body,
        in_specs=[
            pl.BlockSpec(memory_space=pl.ANY),
            pl.BlockSpec(memory_space=pltpu.VMEM),
        ],
        out_specs=pl.BlockSpec(memory_space=pl.ANY),
        out_shape=jax.ShapeDtypeStruct((_N, _P, _D), x.dtype),
        scratch_shapes=[
            pltpu.VMEM((_P, _D), jnp.float32),
            pltpu.VMEM((_K, _P, _D), jnp.float32),
            pltpu.SemaphoreType.DMA,
            pltpu.SemaphoreType.DMA,
            pltpu.SemaphoreType.DMA,
        ],
    )(x3, tok)
    return out3.reshape(_B, _T, _P, _D)


# speedup vs baseline: 1.0504x; 1.0185x over previous
import numpy as np
import jax
import jax.numpy as jnp
from jax.experimental import pallas as pl
from jax.experimental.pallas import tpu as pltpu

_B, _T, _P, _D = 8, 16, 576, 768
_N = _B * _T

_WORDS = [0x8D744451, 0xB39A25C9, 0x587166EB, 0x27893CC9]
_FLAT = np.array([(w >> b) & 1 for w in _WORDS for b in range(32)], dtype=bool)
_MASKED = np.nonzero(_FLAT)[0]
_UNMASKED = np.nonzero(~_FLAT)[0]
_NCP = len(_UNMASKED)
_NTOK = len(_MASKED)

_K = 24
_A = 18


def _body(x_ref, tok_ref, out_ref, tokf, ring, sem_g, sem_s, sem_t):
    tokf[...] = jnp.broadcast_to(tok_ref[0, :], (_P, _D))

    g = [
        pltpu.make_async_copy(x_ref.at[int(f)], ring.at[c % _K], sem_g)
        for c, f in enumerate(_UNMASKED)
    ]
    s = [
        pltpu.make_async_copy(ring.at[c % _K], out_ref.at[int(f)], sem_s)
        for c, f in enumerate(_UNMASKED)
    ]
    t = [pltpu.make_async_copy(tokf, out_ref.at[int(f)], sem_t) for f in _MASKED]

    for c in range(_A):
        g[c].start()
    waited_s = -1
    for c in range(_NCP):
        if c < _NTOK:
            t[c].start()
        g[c].wait()
        s[c].start()
        if c + _A < _NCP:
            if c + _A - _K >= 0:
                s[c + _A - _K].wait()
                waited_s = c + _A - _K
            g[c + _A].start()
    for c in range(waited_s + 1, _NCP):
        s[c].wait()
    for c in range(_NCP, _NTOK):
        t[c].start()
    for c in range(_NTOK):
        t[c].wait()


def kernel(x, mask_token):
    x3 = x.reshape(_N, _P, _D)
    tok = mask_token.reshape(1, _D)
    out3 = pl.pallas_call(
        _body,
        in_specs=[
            pl.BlockSpec(memory_space=pl.ANY),
            pl.BlockSpec(memory_space=pltpu.VMEM),
        ],
        out_specs=pl.BlockSpec(memory_space=pl.ANY),
        out_shape=jax.ShapeDtypeStruct((_N, _P, _D), x.dtype),
        scratch_shapes=[
            pltpu.VMEM((_P, _D), jnp.float32),
            pltpu.VMEM((_K, _P, _D), jnp.float32),
            pltpu.SemaphoreType.DMA,
            pltpu.SemaphoreType.DMA,
            pltpu.SemaphoreType.DMA,
        ],
    )(x3, tok)
    return out3.reshape(_B, _T, _P, _D)
